# baseline (speedup 1.0000x reference)
import functools

import jax
import jax.numpy as jnp
from jax import lax
from jax.experimental import pallas as pl
from jax.experimental.pallas import tpu as pltpu
from jax.experimental.pallas import tpu_sc as plsc

_VOCAB = 1000000
_OUT_DIM = 64
_L = 16


def _body(x_hbm, w_hbm, out_hbm, x_v, idx_v, rows_v, sem):
    cid = lax.axis_index("c")
    sid = lax.axis_index("s")

    @pl.when(jnp.logical_and(cid == 0, sid == 0))
    def _():
        pltpu.sync_copy(x_hbm, x_v)
        xvec = x_v[...]
        lane = lax.iota(jnp.int32, _L)
        for j in range(_OUT_DIM // _L):
            idx_v[pl.ds(j * _L, _L)] = xvec + (lane + j * _L) * _VOCAB
        pltpu.async_copy(w_hbm.at[idx_v], rows_v, sem).wait()
        pltpu.sync_copy(rows_v, out_hbm)


_sc_gather = functools.partial(
    pl.kernel,
    out_type=jax.ShapeDtypeStruct((_OUT_DIM,), jnp.float32),
    mesh=plsc.VectorSubcoreMesh(core_axis_name="c", subcore_axis_name="s"),
    scratch_types=[
        pltpu.VMEM((_L,), jnp.int32),
        pltpu.VMEM((_OUT_DIM,), jnp.int32),
        pltpu.VMEM((_OUT_DIM,), jnp.float32),
        pltpu.SemaphoreType.DMA,
    ],
)(_body)


def kernel(x, W):
    xb = jnp.broadcast_to(x.astype(jnp.int32).reshape(()), (_L,))
    w_flat = W.reshape((_OUT_DIM * _VOCAB,))
    return _sc_gather(xb, w_flat)
